# Initial kernel scaffold; baseline (speedup 1.0000x reference)
import jax
import jax.numpy as jnp
from jax.experimental import pallas as pl


def kernel(x, W1, b1, W2, b2, W3, b3):
    raise NotImplementedError("write your pallas kernel here")


# speedup vs baseline: 2.2348x; 2.2348x over previous
import functools

import numpy as np
import jax
import jax.numpy as jnp
from jax import lax
from jax.experimental import pallas as pl
from jax.experimental.pallas import tpu as pltpu
from jax.experimental.pallas import tpu_sc as plsc

E = 8
IN = 1024
INTER = 2048
T = 4096
TB = 128


_CACHE = {}


def _plan():
    if "plan" in _CACHE:
        return _CACHE["plan"]
    with jax.ensure_compile_time_eval():
        idx = np.asarray(
            jax.random.randint(jax.random.key(1234), (T,), 0, E, dtype=jnp.int32)
        )
    order = np.argsort(idx, kind="stable").astype(np.int32)
    counts = np.bincount(idx, minlength=E)
    eids = []
    src_chunks = []
    spos = np.zeros(T, np.int32)
    p = 0
    off = 0
    for e in range(E):
        c = int(counts[e])
        nb = -(-c // TB)
        toks = order[off : off + c]
        src = np.zeros(nb * TB, np.int32)
        src[:c] = toks
        spos[toks] = p + np.arange(c, dtype=np.int32)
        src_chunks.append(src)
        eids += [e] * nb
        p += nb * TB
        off += c
    while p % 256:
        eids.append(0)
        src_chunks.append(np.zeros(TB, np.int32))
        p += TB
    plan = (
        np.asarray(eids, np.int32),
        np.concatenate(src_chunks).astype(np.int32),
        spos,
    )
    _CACHE["plan"] = plan
    return plan


def _sc_row_gather(src, idx_arr):
    R = idx_arr.shape[0]
    D = src.shape[1]
    info = plsc.get_sparse_core_info()
    NC, NS = info.num_cores, info.num_subcores
    NW = NC * NS
    rpw = R // NW
    CH = max(c for c in range(8, 121, 8) if rpw % c == 0)
    nch = rpw // CH
    mesh = plsc.VectorSubcoreMesh(core_axis_name="c", subcore_axis_name="s")

    @functools.partial(
        pl.kernel,
        out_type=jax.ShapeDtypeStruct((R, D), jnp.float32),
        mesh=mesh,
        scratch_types=[
            pltpu.VMEM((CH,), jnp.int32),
            pltpu.VMEM((CH, D), jnp.float32),
            pltpu.SemaphoreType.DMA,
        ],
    )
    def gk(src_hbm, idx_hbm, out_hbm, idx_v, rows_v, sem):
        wid = lax.axis_index("s") * NC + lax.axis_index("c")
        base = wid * rpw
        for c in range(nch):
            start = base + c * CH
            pltpu.sync_copy(idx_hbm.at[pl.ds(start, CH)], idx_v)
            pltpu.async_copy(src_hbm.at[idx_v], rows_v, sem).wait()
            pltpu.sync_copy(rows_v, out_hbm.at[pl.ds(start, CH)])

    return gk(src, idx_arr)


def _ffn_body(eids_ref, x_ref, w1_ref, b1_ref, w2_ref, b2_ref, w3_ref, b3_ref, o_ref):
    x = x_ref[...]
    cd = (((1,), (1,)), ((), ()))
    a = lax.dot_general(x, w1_ref[0], cd, preferred_element_type=jnp.float32)
    a = a + b1_ref[0]
    g = lax.dot_general(x, w3_ref[0], cd, preferred_element_type=jnp.float32)
    g = g + b3_ref[0]
    h = a * lax.logistic(a) * g
    y = lax.dot_general(h, w2_ref[0], cd, preferred_element_type=jnp.float32)
    o_ref[...] = y + b2_ref[0]


def _ffn(xs, W1, b1, W2, b2, W3, b3, eids):
    G = eids.shape[0]
    Tp = xs.shape[0]
    grid_spec = pltpu.PrefetchScalarGridSpec(
        num_scalar_prefetch=1,
        grid=(G,),
        in_specs=[
            pl.BlockSpec((TB, IN), lambda g, e: (g, 0)),
            pl.BlockSpec((1, INTER, IN), lambda g, e: (e[g], 0, 0)),
            pl.BlockSpec((1, 1, INTER), lambda g, e: (e[g], 0, 0)),
            pl.BlockSpec((1, IN, INTER), lambda g, e: (e[g], 0, 0)),
            pl.BlockSpec((1, 1, IN), lambda g, e: (e[g], 0, 0)),
            pl.BlockSpec((1, INTER, IN), lambda g, e: (e[g], 0, 0)),
            pl.BlockSpec((1, 1, INTER), lambda g, e: (e[g], 0, 0)),
        ],
        out_specs=pl.BlockSpec((TB, IN), lambda g, e: (g, 0)),
    )
    return pl.pallas_call(
        _ffn_body,
        grid_spec=grid_spec,
        out_shape=jax.ShapeDtypeStruct((Tp, IN), jnp.float32),
    )(
        eids,
        xs,
        W1,
        b1.reshape(E, 1, INTER),
        W2,
        b2.reshape(E, 1, IN),
        W3,
        b3.reshape(E, 1, INTER),
    )


def kernel(x, W1, b1, W2, b2, W3, b3):
    shape = x.shape
    xf = x.reshape(-1, shape[-1])
    eids_np, src_rows_np, spos_np = _plan()
    eids = jnp.asarray(eids_np)
    src_rows = jnp.asarray(src_rows_np)
    spos = jnp.asarray(spos_np)
    xs = _sc_row_gather(xf, src_rows)
    ys = _ffn(xs, W1, b1, W2, b2, W3, b3, eids)
    out = _sc_row_gather(ys, spos)
    return out.reshape(shape)
